# baseline (speedup 1.0000x reference)
import functools

import jax
import jax.numpy as jnp
from jax import lax
from jax.experimental import pallas as pl
from jax.experimental.pallas import tpu as pltpu
from jax.experimental.pallas import tpu_sc as plsc

NU = 1_000_000
NI = 100_000
D = 32
F = 16
B = 16384
NC = 2
NS = 16
NW = NC * NS
BPW = B // NW
UPW = NU // NW
IPW = NI // NW
UPW_PAD = ((UPW + 15) // 16) * 16
IPW_PAD = ((IPW + 15) // 16) * 16
CH = 512
NCH = B // CH
PSH = 14
PMASK = (1 << PSH) - 1
INVALID = 0x7FFFFFFF


def _scan_side(ids_v, tab, lp, li, n_loc, n_pad, lo, out_pos, out_id, wid):
    lane = lax.iota(jnp.int32, 16)

    def init_body(j, _):
        tab[pl.ds(j * 16, 16)] = jnp.full((16,), -1, jnp.int32)
        return 0

    lax.fori_loop(0, n_pad // 16, init_body, 0)

    def scan_body(v, _):
        ids16 = ids_v[pl.ds(v * 16, 16)]
        pos16 = lane + v * 16
        lid = ids16 - lo
        m = (lid >= 0) & (lid < n_loc)
        comb = jnp.where(m, (lid << PSH) | pos16, jnp.int32(INVALID))
        ck, _cv = plsc.sort_key_val(comb, pos16)
        lid_s = lax.shift_right_logical(ck, PSH)
        pos_s = ck & PMASK
        valid = ck != jnp.int32(INVALID)
        nxt = jnp.take_along_axis(
            lid_s, jnp.minimum(lane + 1, 15), axis=0, mode="promise_in_bounds"
        )
        win = valid & ((lid_s != nxt) | (lane == 15))
        plsc.store_scatter(tab, [jnp.where(win, lid_s, 0)], pos_s, mask=win)
        return 0

    lax.fori_loop(0, B // 16, scan_body, 0)

    def comp_body(j, carry):
        off, lastc = carry
        pv = tab[pl.ds(j * 16, 16)]
        m = pv >= 0
        cs = plsc.cumsum(jnp.where(m, jnp.int32(1), jnp.int32(0)))
        tot = jnp.max(cs)
        dest = jnp.where(m, off + cs - 1, 0)
        lidv = lane + j * 16
        plsc.store_scatter(
            lp, [lax.shift_right_logical(dest, 9), dest & (CH - 1)], pv, mask=m
        )
        plsc.store_scatter(
            li, [lax.shift_right_logical(dest, 9), dest & (CH - 1)], lidv + lo,
            mask=m,
        )
        packed = jnp.max(jnp.where(m, (lidv << PSH) | pv, jnp.int32(-1)))
        lastc = jnp.where(packed >= 0, packed, lastc)
        return off + tot, lastc

    cnt, lastc = lax.fori_loop(
        0, n_pad // 16, comp_body, (jnp.int32(0), jnp.int32(0))
    )
    cntp = (cnt + (CH - 1)) & jnp.int32(~(CH - 1))
    lastpos = lastc & PMASK
    lastgid = lax.shift_right_logical(lastc, PSH) + lo
    zeros16 = jnp.zeros((16,), jnp.int32)

    def pad_body(j, _):
        dest = cnt + lane + j * 16
        m = dest < cntp
        destc = jnp.where(m, dest, 0)
        idxs = [lax.shift_right_logical(destc, 9), destc & (CH - 1)]
        plsc.store_scatter(lp, idxs, zeros16 + lastpos, mask=m)
        plsc.store_scatter(li, idxs, zeros16 + lastgid, mask=m)
        return 0

    lax.fori_loop(0, CH // 16, pad_body, 0)
    pltpu.sync_copy(lp, out_pos.at[wid])
    pltpu.sync_copy(li, out_id.at[wid])
    return cntp


def _gather_scan_body(
    uids_h, iids_h, cu4_h, ci4_h, ult_h, ilt_h,
    pk_u, pk_i, ult_g, ilt_g, u_pos, u_id, i_pos, i_id, cnts,
    ids_v, idx_s, sidx_s, tab, lp, li, pk_s, tvec, cstage, sem,
):
    wid = lax.axis_index("s") * NC + lax.axis_index("c")
    base = wid * BPW
    lane = lax.iota(jnp.int32, 16)

    def side(ids_h, src4_h, lt_h, lt_g, pk_g):
        pltpu.sync_copy(ids_h, ids_v)
        pltpu.sync_copy(ids_h.at[pl.ds(base, BPW)], idx_s)

        def sidx_body(v, _):
            ids16 = idx_s[pl.ds(v * 16, 16)]
            sidx_s[pl.ds(v * 16, 16)] = lax.shift_right_logical(ids16, 1)
            return 0

        lax.fori_loop(0, BPW // 16, sidx_body, 0)

        def chunk(c, _):
            pltpu.async_copy(
                src4_h.at[sidx_s.at[pl.ds(c * 256, 256)]], pk_s, sem
            ).wait()
            pltpu.sync_copy(pk_s, pk_g.at[pl.ds(base + c * 256, 256)])
            return 0

        lax.fori_loop(0, BPW // 256, chunk, 0)
        pltpu.async_copy(lt_h.at[idx_s], tvec, sem).wait()
        pltpu.sync_copy(tvec, lt_g.at[pl.ds(base, BPW)])

    side(uids_h, cu4_h, ult_h, ult_g, pk_u)
    cu = _scan_side(ids_v, tab, lp, li, UPW, UPW_PAD, wid * UPW, u_pos, u_id, wid)

    side(iids_h, ci4_h, ilt_h, ilt_g, pk_i)
    ci = _scan_side(ids_v, tab, lp, li, IPW, IPW_PAD, wid * IPW, i_pos, i_id, wid)

    cv = jnp.where(lane == 0, cu, jnp.where(lane == 1, ci, 0))
    cstage[...] = cv
    pltpu.sync_copy(cstage.at[pl.ds(0, 8)], cnts.at[wid])


@functools.cache
def _make_gather_scan():
  return pl.kernel(
    _gather_scan_body,
    out_type=(
        jax.ShapeDtypeStruct((B, 4 * D), jnp.float32),
        jax.ShapeDtypeStruct((B, 4 * D), jnp.float32),
        jax.ShapeDtypeStruct((B,), jnp.float32),
        jax.ShapeDtypeStruct((B,), jnp.float32),
        jax.ShapeDtypeStruct((NW, NCH, CH), jnp.int32),
        jax.ShapeDtypeStruct((NW, NCH, CH), jnp.int32),
        jax.ShapeDtypeStruct((NW, NCH, CH), jnp.int32),
        jax.ShapeDtypeStruct((NW, NCH, CH), jnp.int32),
        jax.ShapeDtypeStruct((NW, 8), jnp.int32),
    ),
    mesh=plsc.VectorSubcoreMesh(
        core_axis_name="c", subcore_axis_name="s", num_cores=NC
    ),
    compiler_params=pltpu.CompilerParams(
        needs_layout_passes=False, use_tc_tiling_on_sc=False
    ),
    scratch_types=[
        pltpu.VMEM((B,), jnp.int32),
        pltpu.VMEM((BPW,), jnp.int32),
        pltpu.VMEM((BPW,), jnp.int32),
        pltpu.VMEM((UPW_PAD,), jnp.int32),
        pltpu.VMEM((NCH, CH), jnp.int32),
        pltpu.VMEM((NCH, CH), jnp.int32),
        pltpu.VMEM((256, 4 * D), jnp.float32),
        pltpu.VMEM((BPW,), jnp.float32),
        pltpu.VMEM((16,), jnp.int32),
        pltpu.SemaphoreType.DMA,
    ],
  )


def _scatter_body(
    nu_h, ni_h, ts_h, u_pos, u_id, i_pos, i_id, cnts,
    ue_ref, ie_ref, ut_ref, it_ref,
    upos, uidl, ipos, iidl, rows, tvec, cstage, sem,
):
    wid = lax.axis_index("s") * NC + lax.axis_index("c")
    lane = lax.iota(jnp.int32, 16)
    pltpu.sync_copy(u_pos.at[wid], upos)
    pltpu.sync_copy(u_id.at[wid], uidl)
    pltpu.sync_copy(i_pos.at[wid], ipos)
    pltpu.sync_copy(i_id.at[wid], iidl)
    pltpu.sync_copy(cnts.at[wid], cstage.at[pl.ds(0, 8)])
    cv = cstage[...]
    cu = jnp.max(jnp.where(lane == 0, cv, 0))
    ci = jnp.max(jnp.where(lane == 1, cv, 0))

    def chunk(c, _):
        @pl.when(c * CH < cu)
        def _():
            pltpu.async_copy(nu_h.at[upos.at[c]], rows, sem).wait()
            pltpu.async_copy(rows, ue_ref.at[uidl.at[c]], sem).wait()
            pltpu.async_copy(ts_h.at[upos.at[c]], tvec, sem).wait()
            pltpu.async_copy(tvec, ut_ref.at[uidl.at[c]], sem).wait()

        @pl.when(c * CH < ci)
        def _():
            pltpu.async_copy(ni_h.at[ipos.at[c]], rows, sem).wait()
            pltpu.async_copy(rows, ie_ref.at[iidl.at[c]], sem).wait()
            pltpu.async_copy(ts_h.at[ipos.at[c]], tvec, sem).wait()
            pltpu.async_copy(tvec, it_ref.at[iidl.at[c]], sem).wait()

        return 0

    lax.fori_loop(0, NCH, chunk, 0)


@functools.cache
def _make_scatter():
  return pl.kernel(
    _scatter_body,
    out_type=(),
    mesh=plsc.VectorSubcoreMesh(
        core_axis_name="c", subcore_axis_name="s", num_cores=NC
    ),
    compiler_params=pltpu.CompilerParams(
        needs_layout_passes=False, use_tc_tiling_on_sc=False
    ),
    scratch_types=[
        pltpu.VMEM((NCH, CH), jnp.int32),
        pltpu.VMEM((NCH, CH), jnp.int32),
        pltpu.VMEM((NCH, CH), jnp.int32),
        pltpu.VMEM((NCH, CH), jnp.int32),
        pltpu.VMEM((CH, D), jnp.float32),
        pltpu.VMEM((CH,), jnp.float32),
        pltpu.VMEM((16,), jnp.int32),
        pltpu.SemaphoreType.DMA,
    ],
  )


def _dense_body(
    pk_u, pk_i, uodd, iodd, ult, ilt, ts, feat,
    mue_u, mie_u, mus_u, mis_u, mf_u, vdu_u, vdi_u, b_u,
    mie_i, mue_i, mis_i, mus_i, mf_i, vdi_i, vdu_i, b_i,
    wtu, wti, nu_ref, ni_ref,
):
    hi = jax.lax.Precision.HIGHEST
    uo = uodd[...] > 0
    ue = jnp.where(uo, pk_u[:, 2 * D:3 * D], pk_u[:, 0:D])
    us_ = jnp.where(uo, pk_u[:, 3 * D:4 * D], pk_u[:, D:2 * D])
    io = iodd[...] > 0
    ie = jnp.where(io, pk_i[:, 2 * D:3 * D], pk_i[:, 0:D])
    is_ = jnp.where(io, pk_i[:, 3 * D:4 * D], pk_i[:, D:2 * D])
    du = ts[...] - ult[...]
    di = ts[...] - ilt[...]
    duf = jnp.log1p(jnp.maximum(du, 0.0))
    dif = jnp.log1p(jnp.maximum(di, 0.0))
    uep = ue * (1.0 + du * wtu[...])
    iep = ie * (1.0 + di * wti[...])
    pu = (
        jnp.dot(uep, mue_u[...], precision=hi)
        + jnp.dot(iep, mie_u[...], precision=hi)
        + jnp.dot(us_, mus_u[...], precision=hi)
        + jnp.dot(is_, mis_u[...], precision=hi)
        + jnp.dot(feat[...], mf_u[...], precision=hi)
        + duf * vdu_u[...]
        + dif * vdi_u[...]
        + b_u[...]
    )
    pi = (
        jnp.dot(iep, mie_i[...], precision=hi)
        + jnp.dot(uep, mue_i[...], precision=hi)
        + jnp.dot(is_, mis_i[...], precision=hi)
        + jnp.dot(us_, mus_i[...], precision=hi)
        + jnp.dot(feat[...], mf_i[...], precision=hi)
        + dif * vdi_i[...]
        + duf * vdu_i[...]
        + b_i[...]
    )
    nu = jnp.tanh(pu)
    ni = jnp.tanh(pi)
    nu = nu / jnp.maximum(jnp.sqrt(jnp.sum(nu * nu, axis=1, keepdims=True)), 1e-12)
    ni = ni / jnp.maximum(jnp.sqrt(jnp.sum(ni * ni, axis=1, keepdims=True)), 1e-12)
    nu_ref[...] = nu
    ni_ref[...] = ni


_BS = 1024
_bspec = lambda n: pl.BlockSpec((_BS, n), lambda i: (i, 0))
_wspec = lambda m, n: pl.BlockSpec((m, n), lambda i: (0, 0))

_dense = pl.pallas_call(
    _dense_body,
    grid=(B // _BS,),
    in_specs=[
        _bspec(4 * D), _bspec(4 * D), _bspec(1), _bspec(1),
        _bspec(1), _bspec(1), _bspec(1), _bspec(F),
        _wspec(D, D), _wspec(D, D), _wspec(D, D), _wspec(D, D),
        _wspec(F, D), _wspec(1, D), _wspec(1, D), _wspec(1, D),
        _wspec(D, D), _wspec(D, D), _wspec(D, D), _wspec(D, D),
        _wspec(F, D), _wspec(1, D), _wspec(1, D), _wspec(1, D),
        _wspec(1, D), _wspec(1, D),
    ],
    out_specs=(_bspec(D), _bspec(D)),
    out_shape=(
        jax.ShapeDtypeStruct((B, D), jnp.float32),
        jax.ShapeDtypeStruct((B, D), jnp.float32),
    ),
)


def kernel(user_ids, item_ids, timestamps, features, user_embeddings,
           item_embeddings, user_last_time, item_last_time, user_static,
           item_static, Wt_u, Wt_i, Wih_u, Whh_u, bih_u, bhh_u, Wih_i, Whh_i,
           bih_i, bhh_i):
    uids = user_ids.astype(jnp.int32)
    iids = item_ids.astype(jnp.int32)
    cu4 = jnp.concatenate(
        [user_embeddings, user_static], axis=1).reshape(NU // 2, 4 * D)
    ci4 = jnp.concatenate(
        [item_embeddings, item_static], axis=1).reshape(NI // 2, 4 * D)
    (pk_u, pk_i, ult_g, ilt_g,
     u_pos, u_id, i_pos, i_id, cnts) = _make_gather_scan()(
        uids, iids, cu4, ci4, user_last_time, item_last_time)

    mue_u = (Wih_u[:, 0:D] + Whh_u).T
    mie_u = Wih_u[:, D:2 * D].T
    mus_u = Wih_u[:, 2 * D:3 * D].T
    mis_u = Wih_u[:, 3 * D:4 * D].T
    mf_u = Wih_u[:, 4 * D:4 * D + F].T
    vdu_u = Wih_u[:, 4 * D + F][None, :]
    vdi_u = Wih_u[:, 4 * D + F + 1][None, :]
    b_u = (bih_u + bhh_u)[None, :]
    mie_i = (Wih_i[:, 0:D] + Whh_i).T
    mue_i = Wih_i[:, D:2 * D].T
    mis_i = Wih_i[:, 2 * D:3 * D].T
    mus_i = Wih_i[:, 3 * D:4 * D].T
    mf_i = Wih_i[:, 4 * D:4 * D + F].T
    vdi_i = Wih_i[:, 4 * D + F][None, :]
    vdu_i = Wih_i[:, 4 * D + F + 1][None, :]
    b_i = (bih_i + bhh_i)[None, :]

    new_u, new_i = _dense(
        pk_u, pk_i, (uids & 1)[:, None], (iids & 1)[:, None],
        ult_g[:, None], ilt_g[:, None],
        timestamps[:, None], features,
        mue_u, mie_u, mus_u, mis_u, mf_u, vdu_u, vdi_u, b_u,
        mie_i, mue_i, mis_i, mus_i, mf_i, vdi_i, vdu_i, b_i,
        Wt_u[:, 0][None, :], Wt_i[:, 0][None, :],
    )

    upd_ue = jax.new_ref(user_embeddings)
    upd_ie = jax.new_ref(item_embeddings)
    upd_ut = jax.new_ref(user_last_time)
    upd_it = jax.new_ref(item_last_time)
    _make_scatter()(new_u, new_i, timestamps, u_pos, u_id, i_pos, i_id, cnts,
                    upd_ue, upd_ie, upd_ut, upd_it)
    return (new_u, new_i, jax.freeze(upd_ue), jax.freeze(upd_ie),
            jax.freeze(upd_ut), jax.freeze(upd_it))
